# Initial kernel scaffold; baseline (speedup 1.0000x reference)
import jax
import jax.numpy as jnp
from jax.experimental import pallas as pl


def kernel(input, coords, neighbor, relative_mask, W0, Wg0, b0, bg0, W1, Wg1, b1, bg1, W2, Wg2, b2, bg2, W3, Wg3, b3, bg3, Wf, bf):
    raise NotImplementedError("write your pallas kernel here")


# speedup vs baseline: 39.6849x; 39.6849x over previous
import functools

import jax
import jax.numpy as jnp
from jax import lax
from jax.experimental import pallas as pl
from jax.experimental.pallas import tpu as pltpu

B, N, K, R = 4, 1024, 32, 2
DIMS = [1, 25, 64, 38, 64]
NUM_CLASSES = 10


def _body(x0_ref, coords_ref, coordsT_ref, nbr_ref, rmask_ref,
          w0_ref, wg0_ref, b0_ref, bg0_ref,
          w1_ref, wg1_ref, b1_ref, bg1_ref,
          w2_ref, wg2_ref, b2_ref, bg2_ref,
          w3_ref, wg3_ref, b3_ref, bg3_ref,
          wf_ref, bf_ref,
          out_ref, at0_ref, at1_ref):
    f32 = jnp.float32
    cx = coordsT_ref[0, 0, :][None, :]
    cy = coordsT_ref[0, 1, :][None, :]
    cz = coordsT_ref[0, 2, :][None, :]
    tile_iota = lax.broadcasted_iota(jnp.int32, (8, 1), 0)

    def jt_step(jt, _):
        j0 = pl.multiple_of(jt * 8, 8)
        ct = coords_ref[0, pl.ds(j0, 8), :]
        dx = ct[:, 0][:, None] - cx
        dy = ct[:, 1][:, None] - cy
        dz = ct[:, 2][:, None] - cz
        dist2 = dx * dx + dy * dy + dz * dz + 1e-12
        dist = jnp.sqrt(dist2)
        bf0 = jnp.exp(-dist2)
        bf1 = bf0 * jnp.exp(2.0 * dist - 1.0)
        row_ids = j0 + tile_iota
        e = jnp.zeros((8, N), dtype=f32)
        for k in range(K):
            nbr = nbr_ref[0, k, :][None, :]
            rm = rmask_ref[0, k, :][None, :]
            e = e + jnp.where(row_ids == nbr, rm, 0.0)
        at0_ref[pl.ds(j0, 8), :] = bf0 * e
        at1_ref[pl.ds(j0, 8), :] = bf1 * e
        return 0

    lax.fori_loop(0, N // 8, jt_step, 0)

    inv_k = 1.0 / K

    def layer(z, wt_ref2, wgt_ref2, bc_ref2, bgc_ref2, rows=None):
        a0 = at0_ref[...] if rows is None else at0_ref[:, :rows]
        a1 = at1_ref[...] if rows is None else at1_ref[:, :rows]
        s0 = jnp.dot(z, a0, preferred_element_type=f32)
        s1 = jnp.dot(z, a1, preferred_element_type=f32)
        wt = wt_ref2[...]
        wgt = wgt_ref2[...]
        msg = (jnp.dot(wt[0], s0, preferred_element_type=f32)
               + jnp.dot(wt[1], s1, preferred_element_type=f32)) * inv_k
        gmsg = (jnp.dot(wgt[0], s0, preferred_element_type=f32)
                + jnp.dot(wgt[1], s1, preferred_element_type=f32)) * inv_k
        msg = msg + bc_ref2[...]
        gmsg = gmsg + bgc_ref2[...]
        return jax.nn.relu(msg) * jax.nn.sigmoid(gmsg)

    z = x0_ref[0]
    z = layer(z, w0_ref, wg0_ref, b0_ref, bg0_ref)
    z = layer(z, w1_ref, wg1_ref, b1_ref, bg1_ref)
    z = layer(z, w2_ref, wg2_ref, b2_ref, bg2_ref)
    z = layer(z, w3_ref, wg3_ref, b3_ref, bg3_ref, rows=128)

    pooled = jnp.sum(z[:, :4], axis=1, keepdims=True) * 0.25
    logits = lax.dot_general(pooled, wf_ref[...], (((0,), (0,)), ((), ())),
                             preferred_element_type=f32)
    out_ref[0] = logits + bf_ref[...]


@functools.partial(jax.jit, static_argnames=("interpret",))
def _run(x0, coords, coords_t, nbr_t, rmask_t, weights, interpret=False):
    f32 = jnp.float32
    full = lambda shape: pl.BlockSpec(shape, lambda b: (0,) * len(shape))
    in_specs = [
        pl.BlockSpec((1, 1, N), lambda b: (b, 0, 0)),
        pl.BlockSpec((1, N, 3), lambda b: (b, 0, 0)),
        pl.BlockSpec((1, 3, N), lambda b: (b, 0, 0)),
        pl.BlockSpec((1, K, N), lambda b: (b, 0, 0)),
        pl.BlockSpec((1, K, N), lambda b: (b, 0, 0)),
    ]
    for i in range(4):
        cin, cout = DIMS[i], DIMS[i + 1]
        in_specs += [full((R, cout, cin)), full((R, cout, cin)),
                     full((cout, 1)), full((cout, 1))]
    in_specs += [full((DIMS[-1], NUM_CLASSES)), full((1, NUM_CLASSES))]

    return pl.pallas_call(
        _body,
        grid=(B,),
        in_specs=in_specs,
        out_specs=pl.BlockSpec((1, 1, NUM_CLASSES), lambda b: (b, 0, 0)),
        out_shape=jax.ShapeDtypeStruct((B, 1, NUM_CLASSES), f32),
        scratch_shapes=[pltpu.VMEM((N, N), f32), pltpu.VMEM((N, N), f32)],
        interpret=interpret,
    )(x0, coords, coords_t, nbr_t, rmask_t, *weights)[:, 0, :]


def kernel(input, coords, neighbor, relative_mask,
           W0, Wg0, b0, bg0, W1, Wg1, b1, bg1,
           W2, Wg2, b2, bg2, W3, Wg3, b3, bg3, Wf, bf):
    coords_t = jnp.swapaxes(coords, 1, 2)
    nbr_t = jnp.swapaxes(neighbor.astype(jnp.int32), 1, 2)
    rmask_t = jnp.swapaxes(relative_mask, 1, 2)
    weights = []
    for (w, wg, b, bg) in ((W0, Wg0, b0, bg0), (W1, Wg1, b1, bg1),
                           (W2, Wg2, b2, bg2), (W3, Wg3, b3, bg3)):
        weights += [jnp.swapaxes(w, 1, 2), jnp.swapaxes(wg, 1, 2),
                    b[:, None], bg[:, None]]
    weights += [Wf, bf[None, :]]
    return _run(input, coords, coords_t, nbr_t, rmask_t, tuple(weights))


# speedup vs baseline: 77.4299x; 1.9511x over previous
import functools

import jax
import jax.numpy as jnp
from jax import lax
from jax.experimental import pallas as pl
from jax.experimental.pallas import tpu as pltpu
from jax.experimental.pallas import tpu_sc as plsc

B, N, K, R = 4, 1024, 32, 2
DIMS = [1, 25, 64, 38, 64]
NUM_CLASSES = 10
BN = B * N
NE = BN * K
NW = 32
PPT = BN // NW
EPT = PPT * K
CHK = 128
NCHK = EPT // CHK
PPC = CHK // K
P2T = B * 4 * K
P2PT = P2T // NW

_MESH = plsc.VectorSubcoreMesh(core_axis_name="c", subcore_axis_name="s")
f32 = jnp.float32
i32 = jnp.int32


def _wid():
    return lax.axis_index("s") * 2 + lax.axis_index("c")


def _sc1_body(xc_ref, yc_ref, zc_ref, x0_ref, gidx_ref,
              d2_ref, xg_ref,
              xv, yv, zv, x0v, giv, d2v, xgv):
    wid = _wid()
    base_pt = wid * PPT
    base_e = wid * EPT
    pltpu.sync_copy(xc_ref, xv)
    pltpu.sync_copy(yc_ref, yv)
    pltpu.sync_copy(zc_ref, zv)
    pltpu.sync_copy(x0_ref, x0v)
    pltpu.sync_copy(gidx_ref.at[pl.ds(base_e, EPT)], giv)

    def pt_body(p, _):
        g = base_pt + p
        gs = jnp.full((16,), g, dtype=i32)
        cnx = plsc.load_gather(xv, [gs])
        cny = plsc.load_gather(yv, [gs])
        cnz = plsc.load_gather(zv, [gs])
        for h in range(2):
            off = p * K + h * 16
            idx = giv[pl.ds(off, 16)]
            jx = plsc.load_gather(xv, [idx])
            jy = plsc.load_gather(yv, [idx])
            jz = plsc.load_gather(zv, [idx])
            xj = plsc.load_gather(x0v, [idx])
            dx = jx - cnx
            dy = jy - cny
            dz = jz - cnz
            d2v[pl.ds(off, 16)] = dx * dx + dy * dy + dz * dz
            xgv[pl.ds(off, 16)] = xj
        return 0

    lax.fori_loop(0, PPT, pt_body, 0)
    pltpu.sync_copy(d2v, d2_ref.at[pl.ds(base_e, EPT)])
    pltpu.sync_copy(xgv, xg_ref.at[pl.ds(base_e, EPT)])


_sc1 = functools.partial(
    pl.kernel,
    mesh=_MESH,
    compiler_params=pltpu.CompilerParams(needs_layout_passes=False, use_tc_tiling_on_sc=False),
    out_type=(jax.ShapeDtypeStruct((NE,), f32),
              jax.ShapeDtypeStruct((NE,), f32)),
    scratch_types=[
        pltpu.VMEM((BN,), f32), pltpu.VMEM((BN,), f32),
        pltpu.VMEM((BN,), f32), pltpu.VMEM((BN,), f32),
        pltpu.VMEM((EPT,), i32),
        pltpu.VMEM((EPT,), f32), pltpu.VMEM((EPT,), f32),
    ],
)(_sc1_body)


def _sc2_body(z1_ref, gidx2_ref, b0_ref, b1_ref,
              s1_ref,
              gi2v, b0v, b1v, rows0, rows1, s1v, sem0, sem1):
    wid = _wid()
    pltpu.sync_copy(gidx2_ref.at[pl.ds(wid * NCHK, NCHK)], gi2v)
    pltpu.sync_copy(b0_ref.at[pl.ds(wid * EPT, EPT)], b0v)
    pltpu.sync_copy(b1_ref.at[pl.ds(wid * EPT, EPT)], b1v)

    rows = (rows0, rows1)
    sems = (sem0, sem1)
    handles = [None, None]
    handles[0] = pltpu.async_copy(z1_ref.at[gi2v.at[0]], rows0, sem0)
    for c in range(NCHK):
        if c + 1 < NCHK:
            handles[(c + 1) % 2] = pltpu.async_copy(
                z1_ref.at[gi2v.at[c + 1]], rows[(c + 1) % 2],
                sems[(c + 1) % 2])
        handles[c % 2].wait()
        rv = rows[c % 2]

        def pt_body(pp, _):
            p = c * PPC + pp
            zero = jnp.zeros((16,), f32)

            def e_body(e8, carry):
                a00, a01, a10, a11 = carry
                for u in range(4):
                    e = e8 * 4 + u
                    r = pp * K + e
                    lo = rv[r, 0:16]
                    hi = rv[r, 16:32]
                    es = jnp.full((16,), p * K + e, dtype=i32)
                    b0s = plsc.load_gather(b0v, [es])
                    b1s = plsc.load_gather(b1v, [es])
                    a00 = a00 + lo * b0s
                    a01 = a01 + hi * b0s
                    a10 = a10 + lo * b1s
                    a11 = a11 + hi * b1s
                return (a00, a01, a10, a11)

            a00, a01, a10, a11 = lax.fori_loop(
                0, K // 4, e_body, (zero, zero, zero, zero))
            s1v[p, 0:16] = a00
            s1v[p, 16:32] = a01
            s1v[p, 32:48] = a10
            s1v[p, 48:64] = a11
            return 0

        lax.fori_loop(0, PPC, pt_body, 0)
    pltpu.sync_copy(s1v, s1_ref.at[pl.ds(wid * PPT, PPT)])


_sc2 = functools.partial(
    pl.kernel,
    mesh=_MESH,
    compiler_params=pltpu.CompilerParams(needs_layout_passes=False, use_tc_tiling_on_sc=False),
    out_type=jax.ShapeDtypeStruct((BN, 2 * 32), f32),
    scratch_types=[
        pltpu.VMEM((NCHK, CHK), i32),
        pltpu.VMEM((EPT,), f32), pltpu.VMEM((EPT,), f32),
        pltpu.VMEM((CHK, 32), f32), pltpu.VMEM((CHK, 32), f32),
        pltpu.VMEM((PPT, 64), f32),
        pltpu.SemaphoreType.DMA, pltpu.SemaphoreType.DMA,
    ],
)(_sc2_body)


def _sc3_body(z2_ref, gidxk_ref, b02_ref, b12_ref, pidx_ref,
              s2_ref,
              pidv, girows, b0r, b1r, zr0, zr1, s2v, sema, sem0, sem1):
    wid = _wid()
    pltpu.sync_copy(pidx_ref.at[pl.ds(wid * P2PT, P2PT)], pidv)
    pltpu.async_copy(gidxk_ref.at[pidv], girows, sema).wait()
    pltpu.async_copy(b02_ref.at[pidv], b0r, sema).wait()
    pltpu.async_copy(b12_ref.at[pidv], b1r, sema).wait()

    zrs = (zr0, zr1)
    sems = (sem0, sem1)
    handles = [None, None]
    handles[0] = pltpu.async_copy(z2_ref.at[girows.at[0]], zr0, sem0)
    for q in range(P2PT):
        if q + 1 < P2PT:
            handles[(q + 1) % 2] = pltpu.async_copy(
                z2_ref.at[girows.at[q + 1]], zrs[(q + 1) % 2],
                sems[(q + 1) % 2])
        handles[q % 2].wait()
        rv = zrs[q % 2]
        zero = jnp.zeros((16,), f32)

        def e_body(e2, carry):
            accs = list(carry)
            for u in range(2):
                e = e2 * 2 + u
                qs = jnp.full((16,), q, dtype=i32)
                es = jnp.full((16,), e, dtype=i32)
                b0s = plsc.load_gather(b0r, [qs, es])
                b1s = plsc.load_gather(b1r, [qs, es])
                for seg in range(4):
                    v = rv[e, pl.ds(seg * 16, 16)]
                    accs[seg] = accs[seg] + v * b0s
                    accs[4 + seg] = accs[4 + seg] + v * b1s
            return tuple(accs)

        accs = lax.fori_loop(0, K // 2, e_body, (zero,) * 8)
        for seg in range(8):
            s2v[q, pl.ds(seg * 16, 16)] = accs[seg]
    pltpu.sync_copy(s2v, s2_ref.at[pl.ds(wid * P2PT, P2PT)])


_sc3 = functools.partial(
    pl.kernel,
    mesh=_MESH,
    compiler_params=pltpu.CompilerParams(needs_layout_passes=False, use_tc_tiling_on_sc=False),
    out_type=jax.ShapeDtypeStruct((P2T, 2 * 64), f32),
    scratch_types=[
        pltpu.VMEM((P2PT,), i32),
        pltpu.VMEM((P2PT, K), i32),
        pltpu.VMEM((P2PT, K), f32), pltpu.VMEM((P2PT, K), f32),
        pltpu.VMEM((K, 64), f32), pltpu.VMEM((K, 64), f32),
        pltpu.VMEM((P2PT, 2 * 64), f32),
        pltpu.SemaphoreType.DMA,
        pltpu.SemaphoreType.DMA, pltpu.SemaphoreType.DMA,
    ],
)(_sc3_body)


def _tc1_body(d2_ref, rm_ref, xg_ref, w00_ref, w01_ref, wg00_ref, wg01_ref,
              b0_ref, bg0_ref, bas0_ref, bas1_ref, z1_ref):
    d2 = d2_ref[...] + 1e-12
    dist = jnp.sqrt(d2)
    rm = rm_ref[...]
    bas0 = jnp.exp(-d2) * rm
    bas1 = bas0 * jnp.exp(2.0 * dist - 1.0)
    bas0_ref[...] = bas0
    bas1_ref[...] = bas1
    xg = xg_ref[...]
    inv_k = 1.0 / K
    s0 = jnp.sum(bas0 * xg, axis=1, keepdims=True) * inv_k
    s1 = jnp.sum(bas1 * xg, axis=1, keepdims=True) * inv_k
    msg = s0 * w00_ref[...] + s1 * w01_ref[...] + b0_ref[...]
    gmsg = s0 * wg00_ref[...] + s1 * wg01_ref[...] + bg0_ref[...]
    z1 = jax.nn.relu(msg) * jax.nn.sigmoid(gmsg)
    z1_ref[...] = jnp.concatenate(
        [z1, jnp.zeros((BN, 32 - DIMS[1]), f32)], axis=1)


def _tc2_body(s1_ref, w_ref, wg_ref, b_ref, bg_ref, z2_ref):
    s1 = s1_ref[...] * (1.0 / K)
    msg = jnp.dot(s1, w_ref[...], preferred_element_type=f32) + b_ref[...]
    gmsg = jnp.dot(s1, wg_ref[...], preferred_element_type=f32) + bg_ref[...]
    z2_ref[...] = jax.nn.relu(msg) * jax.nn.sigmoid(gmsg)


def _tc3_body(s2_ref, w2_ref, wg2_ref, b2_ref, bg2_ref,
              b0t_ref, b1t_ref, w3_ref, wg3_ref, b3_ref, bg3_ref,
              wf_ref, bf_ref, out_ref):
    inv_k = 1.0 / K
    s2 = s2_ref[...] * inv_k
    msg = jnp.dot(s2, w2_ref[...], preferred_element_type=f32) + b2_ref[...]
    gmsg = jnp.dot(s2, wg2_ref[...], preferred_element_type=f32) + bg2_ref[...]
    z3 = jax.nn.relu(msg) * jax.nn.sigmoid(gmsg)

    rows0 = []
    rows1 = []
    for q in range(16):
        blk = z3[q * K:(q + 1) * K, :]
        w0c = b0t_ref[:, q:q + 1]
        w1c = b1t_ref[:, q:q + 1]
        rows0.append(jnp.sum(blk * w0c, axis=0, keepdims=True))
        rows1.append(jnp.sum(blk * w1c, axis=0, keepdims=True))
    s3 = jnp.concatenate(
        [jnp.concatenate(rows0, axis=0),
         jnp.concatenate(rows1, axis=0)], axis=1) * inv_k
    msg3 = jnp.dot(s3, w3_ref[...], preferred_element_type=f32) + b3_ref[...]
    gmsg3 = jnp.dot(s3, wg3_ref[...], preferred_element_type=f32) + bg3_ref[...]
    out4 = jax.nn.relu(msg3) * jax.nn.sigmoid(gmsg3)

    ri = lax.broadcasted_iota(i32, (4, 16), 0)
    ci = lax.broadcasted_iota(i32, (4, 16), 1)
    pmat = jnp.where(ci // 4 == ri, 0.25, 0.0).astype(f32)
    pooled = jnp.dot(pmat, out4, preferred_element_type=f32)
    out_ref[...] = (jnp.dot(pooled, wf_ref[...], preferred_element_type=f32)
                    + bf_ref[...])


@jax.jit
def _forward_impl(xc, yc, zc, x0, gidx, gidx2, gidxk, rmask2,
                  w00, w01, wg00, wg01, b0r, bg0r,
                  wcat1, wgcat1, b1r, bg1r,
                  wcat2, wgcat2, b2r, bg2r,
                  pidx, w3f, wg3f, b3r, bg3r, wf, bfr):
    d2e, xge = _sc1(xc, yc, zc, x0, gidx)

    bas0, bas1, z1 = pl.pallas_call(
        _tc1_body,
        out_shape=(jax.ShapeDtypeStruct((BN, K), f32),
                   jax.ShapeDtypeStruct((BN, K), f32),
                   jax.ShapeDtypeStruct((BN, 32), f32)),
    )(d2e.reshape(BN, K), rmask2, xge.reshape(BN, K),
      w00, w01, wg00, wg01, b0r, bg0r)

    s1 = _sc2(z1, gidx2, bas0.reshape(NE), bas1.reshape(NE))

    z2 = pl.pallas_call(
        _tc2_body,
        out_shape=jax.ShapeDtypeStruct((BN, 64), f32),
    )(s1, wcat1, wgcat1, b1r, bg1r)

    s2 = _sc3(z2, gidxk, bas0, bas1, pidx)

    b0t = bas0.reshape(B, N, K)[:, :4, :].reshape(16, K).T
    b1t = bas1.reshape(B, N, K)[:, :4, :].reshape(16, K).T

    out = pl.pallas_call(
        _tc3_body,
        out_shape=jax.ShapeDtypeStruct((B, NUM_CLASSES), f32),
    )(s2, wcat2, wgcat2, b2r, bg2r, b0t, b1t,
      w3f, wg3f, b3r, bg3r, wf, bfr)
    return out


def kernel(input, coords, neighbor, relative_mask,
           W0, Wg0, b0, bg0, W1, Wg1, b1, bg1,
           W2, Wg2, b2, bg2, W3, Wg3, b3, bg3, Wf, bf):
    xc = coords[..., 0].reshape(BN)
    yc = coords[..., 1].reshape(BN)
    zc = coords[..., 2].reshape(BN)
    x0 = input[:, 0, :].reshape(BN)
    nbr = neighbor.astype(i32)
    gidx = (nbr + (jnp.arange(B, dtype=i32) * N)[:, None, None]).reshape(NE)
    gidx2 = gidx.reshape(NE // CHK, CHK)
    gidxk = gidx.reshape(BN, K)
    pidx = gidx.reshape(B, N, K)[:, :4, :].reshape(P2T)
    rmask2 = relative_mask.reshape(BN, K)

    z64 = jnp.zeros((64, 64), dtype=f32)
    wcat1 = z64.at[0:25, :].set(W1[0]).at[32:57, :].set(W1[1])
    wgcat1 = z64.at[0:25, :].set(Wg1[0]).at[32:57, :].set(Wg1[1])
    wcat2 = jnp.concatenate([W2[0], W2[1]], axis=0)
    wgcat2 = jnp.concatenate([Wg2[0], Wg2[1]], axis=0)
    w3f = jnp.concatenate([W3[0], W3[1]], axis=0)
    wg3f = jnp.concatenate([Wg3[0], Wg3[1]], axis=0)

    return _forward_impl(
        xc, yc, zc, x0, gidx, gidx2, gidxk, rmask2,
        W0[0], W0[1], Wg0[0], Wg0[1], b0[None, :], bg0[None, :],
        wcat1, wgcat1, b1[None, :], bg1[None, :],
        wcat2, wgcat2, b2[None, :], bg2[None, :],
        pidx, w3f, wg3f, b3[None, :], bg3[None, :], Wf, bf[None, :])
